# baseline (speedup 1.0000x reference)
import functools

import jax
import jax.numpy as jnp
from jax import lax
from jax.experimental import pallas as pl
from jax.experimental.pallas import tpu as pltpu
from jax.experimental.pallas import tpu_sc as plsc

_N = 10000
_D = 128
_E = 320000
_S1 = 25
_S2 = 10

_NW = 32
_PW = 320
_NP = _NW * _PW


_NBUF = 4


def _make_gmax(S, G):
    K = G * S
    chunks = _PW // G
    nquads = chunks // _NBUF
    assert chunks % _NBUF == 0 and K <= 128
    mesh = plsc.VectorSubcoreMesh(core_axis_name="c", subcore_axis_name="s")

    @functools.partial(
        pl.kernel,
        mesh=mesh,
        out_type=jax.ShapeDtypeStruct((_NP, _D), jnp.float32),
        scratch_types=[
            pltpu.VMEM((chunks, K), jnp.int32),
            pltpu.VMEM((_NBUF, K, _D), jnp.float32),
            pltpu.VMEM((_PW, _D), jnp.float32),
        ]
        + [pltpu.SemaphoreType.DMA] * _NBUF,
    )
    def gmax(y_hbm, idx_hbm, out_hbm, idx_v, bufs, outv, *sems):
        wid = lax.axis_index("s") * 2 + lax.axis_index("c")
        pltpu.sync_copy(idx_hbm.at[pl.ds(wid * chunks, chunks)], idx_v)
        for b in range(_NBUF):
            pltpu.async_copy(y_hbm.at[idx_v.at[b]], bufs.at[b], sems[b])

        nl = _D // 16

        def compute(b, c):
            for g in range(G):
                base = g * S
                acc = [bufs[b, base, pl.ds(l * 16, 16)] for l in range(nl)]
                for s in range(1, S):
                    for l in range(nl):
                        acc[l] = jnp.maximum(acc[l], bufs[b, base + s, pl.ds(l * 16, 16)])
                for l in range(nl):
                    outv[c * G + g, pl.ds(l * 16, 16)] = acc[l]

        def body(q, carry):
            c0 = _NBUF * q
            for b in range(_NBUF):
                pltpu.make_async_copy(y_hbm.at[idx_v.at[c0 + b]], bufs.at[b], sems[b]).wait()
                compute(b, c0 + b)

                @pl.when(q + 1 < nquads)
                def _():
                    pltpu.async_copy(
                        y_hbm.at[idx_v.at[c0 + b + _NBUF]], bufs.at[b], sems[b])

            return carry

        lax.fori_loop(0, nquads, body, 0)
        pltpu.sync_copy(outv, out_hbm.at[pl.ds(wid * _PW, _PW)])

    return gmax


_gmax_cache = {}


def _gmax(S, G):
    if (S, G) not in _gmax_cache:
        _gmax_cache[(S, G)] = _make_gmax(S, G)
    return _gmax_cache[(S, G)]


def _pack_idx(neigh, S, K):
    flat = neigh.reshape(-1)
    pad = _NP * S - flat.shape[0]
    flat = jnp.concatenate([flat, jnp.zeros((pad,), jnp.int32)])
    return flat.reshape(-1, K)


_BR = 2000


def _relu_mm_body(x_ref, wt_ref, b_ref, y_ref):
    y_ref[...] = jnp.maximum(
        jnp.dot(x_ref[...], wt_ref[...], preferred_element_type=jnp.float32)
        + b_ref[...],
        0.0,
    )


def _post_body(x_ref, agg_ref, wat_ref, wbt_ref, o_ref):
    o = jnp.dot(x_ref[...], wat_ref[...], preferred_element_type=jnp.float32)
    o = o + jnp.dot(agg_ref[...], wbt_ref[...], preferred_element_type=jnp.float32)
    l2 = jnp.sqrt(jnp.sum(o * o, axis=1, keepdims=True))
    o_ref[...] = o / l2


def _post_y_body(x_ref, agg_ref, wat_ref, wbt_ref, waggt_ref, b_ref, h_ref, y_ref):
    o = jnp.dot(x_ref[...], wat_ref[...], preferred_element_type=jnp.float32)
    o = o + jnp.dot(agg_ref[...], wbt_ref[...], preferred_element_type=jnp.float32)
    l2 = jnp.sqrt(jnp.sum(o * o, axis=1, keepdims=True))
    h = o / l2
    h_ref[...] = h
    y_ref[...] = jnp.maximum(
        jnp.dot(h, waggt_ref[...], preferred_element_type=jnp.float32) + b_ref[...],
        0.0,
    )


def _row_spec():
    return pl.BlockSpec((_BR, _D), lambda i: (i, 0))


def _full_spec():
    return pl.BlockSpec((_D, _D), lambda i: (0, 0))


def _bias_spec():
    return pl.BlockSpec((1, _D), lambda i: (0, 0))


def _mm_relu(x, wt, b):
    return pl.pallas_call(
        _relu_mm_body,
        grid=(_N // _BR,),
        in_specs=[_row_spec(), _full_spec(), _bias_spec()],
        out_specs=_row_spec(),
        out_shape=jax.ShapeDtypeStruct((_N, _D), jnp.float32),
    )(x, wt, b)


def _post(x, agg, wat, wbt):
    return pl.pallas_call(
        _post_body,
        grid=(_N // _BR,),
        in_specs=[_row_spec(), _row_spec(), _full_spec(), _full_spec()],
        out_specs=_row_spec(),
        out_shape=jax.ShapeDtypeStruct((_N, _D), jnp.float32),
    )(x, agg, wat, wbt)


def _post_y(x, agg, wat, wbt, waggt, b):
    return pl.pallas_call(
        _post_y_body,
        grid=(_N // _BR,),
        in_specs=[_row_spec(), _row_spec(), _full_spec(), _full_spec(),
                  _full_spec(), _bias_spec()],
        out_specs=[_row_spec(), _row_spec()],
        out_shape=[jax.ShapeDtypeStruct((_N, _D), jnp.float32),
                   jax.ShapeDtypeStruct((_N, _D), jnp.float32)],
    )(x, agg, wat, wbt, waggt, b)


def _sample(key, S, deg, offsets, dst_sorted):
    u = jax.random.uniform(key, (_N, S))
    degf = jnp.maximum(deg, 1).astype(jnp.float32)
    idx = jnp.floor(u * degf[:, None]).astype(jnp.int32)
    pos = jnp.clip(offsets[:, None] + idx, 0, dst_sorted.shape[0] - 1)
    return dst_sorted[pos]


def kernel(x, edge_index, W_agg1, b_agg1, W1, W_agg2, b_agg2, W2):
    src = jnp.concatenate([edge_index[0], edge_index[1]])
    dst = jnp.concatenate([edge_index[1], edge_index[0]])
    order = jnp.argsort(src)
    dst_sorted = dst[order]
    deg = jnp.bincount(src, length=_N)
    offsets = jnp.cumsum(deg) - deg

    key = jax.random.key(42)
    k1, k2 = jax.random.split(key)
    idx1 = _pack_idx(_sample(k1, _S1, deg, offsets, dst_sorted), _S1, 4 * _S1)
    idx2 = _pack_idx(_sample(k2, _S2, deg, offsets, dst_sorted), _S2, 8 * _S2)

    y1 = _mm_relu(x, W_agg1.T, b_agg1[None])
    agg1 = _gmax(_S1, 4)(y1, idx1)[:_N]
    h1, y2 = _post_y(x, agg1, W1[:, :_D].T, W1[:, _D:].T, W_agg2.T, b_agg2[None])
    agg2 = _gmax(_S2, 8)(y2, idx2)[:_N]
    h2 = _post(h1, agg2, W2[:, :_D].T, W2[:, _D:].T)
    return h2


# speedup vs baseline: 1.2877x; 1.2877x over previous
import functools

import jax
import jax.numpy as jnp
from jax import lax
from jax.experimental import pallas as pl
from jax.experimental.pallas import tpu as pltpu
from jax.experimental.pallas import tpu_sc as plsc

_N = 10000
_D = 128
_E = 320000
_S1 = 25
_S2 = 10

_NW = 32
_PW = 320
_NP = _NW * _PW


_NBUF = 4


def _make_gmax(S, G):
    K = G * S
    chunks = _PW // G
    nquads = chunks // _NBUF
    assert chunks % _NBUF == 0 and K <= 128
    mesh = plsc.VectorSubcoreMesh(core_axis_name="c", subcore_axis_name="s")

    @functools.partial(
        pl.kernel,
        mesh=mesh,
        out_type=jax.ShapeDtypeStruct((_NP, _D), jnp.float32),
        scratch_types=[
            pltpu.VMEM((chunks, K), jnp.int32),
            pltpu.VMEM((_NBUF, K, _D), jnp.float32),
            pltpu.VMEM((_PW, _D), jnp.float32),
        ]
        + [pltpu.SemaphoreType.DMA] * _NBUF,
    )
    def gmax(y_hbm, idx_hbm, out_hbm, idx_v, bufs, outv, *sems):
        wid = lax.axis_index("s") * 2 + lax.axis_index("c")
        pltpu.sync_copy(idx_hbm.at[pl.ds(wid * chunks, chunks)], idx_v)
        for b in range(_NBUF):
            pltpu.async_copy(y_hbm.at[idx_v.at[b]], bufs.at[b], sems[b])

        nl = _D // 16

        def compute(b, c):
            for g in range(G):
                base = g * S
                acc = [bufs[b, base, pl.ds(l * 16, 16)] for l in range(nl)]
                for s in range(1, S):
                    for l in range(nl):
                        acc[l] = jnp.maximum(acc[l], bufs[b, base + s, pl.ds(l * 16, 16)])
                for l in range(nl):
                    outv[c * G + g, pl.ds(l * 16, 16)] = acc[l]

        def body(q, carry):
            c0 = _NBUF * q
            for b in range(_NBUF):
                pltpu.make_async_copy(y_hbm.at[idx_v.at[c0 + b]], bufs.at[b], sems[b]).wait()
                compute(b, c0 + b)

                @pl.when(q + 1 < nquads)
                def _():
                    pltpu.async_copy(
                        y_hbm.at[idx_v.at[c0 + b + _NBUF]], bufs.at[b], sems[b])

            return carry

        lax.fori_loop(0, nquads, body, 0)
        pltpu.sync_copy(outv, out_hbm.at[pl.ds(wid * _PW, _PW)])

    return gmax


_gmax_cache = {}


def _gmax(S, G):
    if (S, G) not in _gmax_cache:
        _gmax_cache[(S, G)] = _make_gmax(S, G)
    return _gmax_cache[(S, G)]


def _pack_idx(neigh, S, K):
    flat = neigh.reshape(-1)
    pad = _NP * S - flat.shape[0]
    flat = jnp.concatenate([flat, jnp.arange(pad, dtype=jnp.int32) % _N])
    return flat.reshape(-1, K)


_BR = 2000


def _relu_mm_body(x_ref, wt_ref, b_ref, y_ref):
    y_ref[...] = jnp.maximum(
        jnp.dot(x_ref[...], wt_ref[...], preferred_element_type=jnp.float32)
        + b_ref[...],
        0.0,
    )


def _post_body(x_ref, agg_ref, wat_ref, wbt_ref, o_ref):
    o = jnp.dot(x_ref[...], wat_ref[...], preferred_element_type=jnp.float32)
    o = o + jnp.dot(agg_ref[...], wbt_ref[...], preferred_element_type=jnp.float32)
    l2 = jnp.sqrt(jnp.sum(o * o, axis=1, keepdims=True))
    o_ref[...] = o / l2


def _post_y_body(x_ref, agg_ref, wat_ref, wbt_ref, waggt_ref, b_ref, h_ref, y_ref):
    o = jnp.dot(x_ref[...], wat_ref[...], preferred_element_type=jnp.float32)
    o = o + jnp.dot(agg_ref[...], wbt_ref[...], preferred_element_type=jnp.float32)
    l2 = jnp.sqrt(jnp.sum(o * o, axis=1, keepdims=True))
    h = o / l2
    h_ref[...] = h
    y_ref[...] = jnp.maximum(
        jnp.dot(h, waggt_ref[...], preferred_element_type=jnp.float32) + b_ref[...],
        0.0,
    )


def _row_spec():
    return pl.BlockSpec((_BR, _D), lambda i: (i, 0))


def _full_spec():
    return pl.BlockSpec((_D, _D), lambda i: (0, 0))


def _bias_spec():
    return pl.BlockSpec((1, _D), lambda i: (0, 0))


def _mm_relu(x, wt, b):
    return pl.pallas_call(
        _relu_mm_body,
        grid=(_N // _BR,),
        in_specs=[_row_spec(), _full_spec(), _bias_spec()],
        out_specs=_row_spec(),
        out_shape=jax.ShapeDtypeStruct((_N, _D), jnp.float32),
    )(x, wt, b)


def _post(x, agg, wat, wbt):
    return pl.pallas_call(
        _post_body,
        grid=(_N // _BR,),
        in_specs=[_row_spec(), _row_spec(), _full_spec(), _full_spec()],
        out_specs=_row_spec(),
        out_shape=jax.ShapeDtypeStruct((_N, _D), jnp.float32),
    )(x, agg, wat, wbt)


def _post_y(x, agg, wat, wbt, waggt, b):
    return pl.pallas_call(
        _post_y_body,
        grid=(_N // _BR,),
        in_specs=[_row_spec(), _row_spec(), _full_spec(), _full_spec(),
                  _full_spec(), _bias_spec()],
        out_specs=[_row_spec(), _row_spec()],
        out_shape=[jax.ShapeDtypeStruct((_N, _D), jnp.float32),
                   jax.ShapeDtypeStruct((_N, _D), jnp.float32)],
    )(x, agg, wat, wbt, waggt, b)


def _sample(key, S, deg, offsets, dst_sorted):
    u = jax.random.uniform(key, (_N, S))
    degf = jnp.maximum(deg, 1).astype(jnp.float32)
    idx = jnp.floor(u * degf[:, None]).astype(jnp.int32)
    pos = jnp.clip(offsets[:, None] + idx, 0, dst_sorted.shape[0] - 1)
    return dst_sorted[pos]


def kernel(x, edge_index, W_agg1, b_agg1, W1, W_agg2, b_agg2, W2):
    src = jnp.concatenate([edge_index[0], edge_index[1]])
    dst = jnp.concatenate([edge_index[1], edge_index[0]])
    order = jnp.argsort(src)
    dst_sorted = dst[order]
    deg = jnp.bincount(src, length=_N)
    offsets = jnp.cumsum(deg) - deg

    key = jax.random.key(42)
    k1, k2 = jax.random.split(key)
    idx1 = _pack_idx(_sample(k1, _S1, deg, offsets, dst_sorted), _S1, 4 * _S1)
    idx2 = _pack_idx(_sample(k2, _S2, deg, offsets, dst_sorted), _S2, 8 * _S2)

    y1 = _mm_relu(x, W_agg1.T, b_agg1[None])
    agg1 = _gmax(_S1, 4)(y1, idx1)[:_N]
    h1, y2 = _post_y(x, agg1, W1[:, :_D].T, W1[:, _D:].T, W_agg2.T, b_agg2[None])
    agg2 = _gmax(_S2, 8)(y2, idx2)[:_N]
    h2 = _post(h1, agg2, W2[:, :_D].T, W2[:, _D:].T)
    return h2
